# Initial kernel scaffold; baseline (speedup 1.0000x reference)
import functools

import jax
import jax.numpy as jnp
from jax import lax
from jax.experimental import pallas as pl
from jax.experimental.pallas import tpu as pltpu
from jax.experimental.pallas import tpu_sc as plsc

NUM_SEGMENTS = 4096
D = 128
L = 16
NC = 2
NS = 16
NW = NC * NS
CHUNK = 200
SUB = 100
SEG_T = NUM_SEGMENTS // NS
SEG_W = NUM_SEGMENTS // NW


def _partial_sums(atom_fea, idx3):
    n = atom_fea.shape[0]
    rows_per_w = n // NW
    n_chunks = rows_per_w // CHUNK
    mesh = plsc.VectorSubcoreMesh(core_axis_name="c", subcore_axis_name="s")

    @functools.partial(
        pl.kernel,
        mesh=mesh,
        out_type=[
            jax.ShapeDtypeStruct((NC, NUM_SEGMENTS, D), jnp.float32),
            jax.ShapeDtypeStruct((NC, NUM_SEGMENTS, L), jnp.float32),
        ],
        scratch_types=[
            pltpu.VMEM((CHUNK, D), jnp.float32),
            pltpu.VMEM((CHUNK // SUB, SUB), jnp.int32),
            pltpu.VMEM((SUB, L), jnp.float32),
            pltpu.VMEM((SEG_T, D), jnp.float32),
            pltpu.VMEM((SEG_T, L), jnp.float32),
            pltpu.VMEM_SHARED((NUM_SEGMENTS, D), jnp.float32),
            pltpu.VMEM_SHARED((NUM_SEGMENTS, L), jnp.float32),
        ],
    )
    def k1(fea_hbm, idx_hbm, psums_hbm, pcnts_hbm,
           rows_v, idx_v, ones_v, seg_v, cnt_v, spm_sums, spm_cnts):
        cid = lax.axis_index("c")
        sid = lax.axis_index("s")
        wid = cid * NS + sid

        zeros16 = jnp.zeros((L,), jnp.float32)
        ones16 = jnp.ones((L,), jnp.float32)

        def zrow(r, carry):
            for j in range(D // L):
                seg_v[r, pl.ds(j * L, L)] = zeros16
            cnt_v[r] = zeros16
            return carry
        lax.fori_loop(0, SEG_T, zrow, 0)

        def orow(r, carry):
            ones_v[r] = ones16
            return carry
        lax.fori_loop(0, SUB, orow, 0)

        pltpu.sync_copy(seg_v, spm_sums.at[pl.ds(sid * SEG_T, SEG_T)])
        pltpu.sync_copy(cnt_v, spm_cnts.at[pl.ds(sid * SEG_T, SEG_T)])
        plsc.subcore_barrier()

        base_row = wid * rows_per_w
        base_chunk = wid * n_chunks

        def body(i, carry):
            pltpu.sync_copy(fea_hbm.at[pl.ds(base_row + i * CHUNK, CHUNK)],
                            rows_v)
            pltpu.sync_copy(idx_hbm.at[base_chunk + i], idx_v)
            for j in range(CHUNK // SUB):
                pltpu.sync_copy(rows_v.at[pl.ds(j * SUB, SUB)],
                                spm_sums.at[idx_v.at[j]], add=True)
                pltpu.sync_copy(ones_v, spm_cnts.at[idx_v.at[j]], add=True)
            return carry
        lax.fori_loop(0, n_chunks, body, 0)

        plsc.subcore_barrier()

        pltpu.sync_copy(spm_sums.at[pl.ds(sid * SEG_T, SEG_T)], seg_v)
        pltpu.sync_copy(seg_v, psums_hbm.at[cid, pl.ds(sid * SEG_T, SEG_T)])
        pltpu.sync_copy(spm_cnts.at[pl.ds(sid * SEG_T, SEG_T)], cnt_v)
        pltpu.sync_copy(cnt_v, pcnts_hbm.at[cid, pl.ds(sid * SEG_T, SEG_T)])

    return k1(atom_fea, idx3)


def _combine(psums, pcnts):
    mesh = plsc.VectorSubcoreMesh(core_axis_name="c", subcore_axis_name="s")

    @functools.partial(
        pl.kernel,
        mesh=mesh,
        out_type=jax.ShapeDtypeStruct((NUM_SEGMENTS, D), jnp.float32),
        scratch_types=[
            pltpu.VMEM((SEG_W, D), jnp.float32),
            pltpu.VMEM((SEG_W, D), jnp.float32),
            pltpu.VMEM((SEG_W, L), jnp.float32),
            pltpu.VMEM((SEG_W, L), jnp.float32),
        ],
    )
    def k2(psums_hbm, pcnts_hbm, out_hbm, a_v, b_v, ca_v, cb_v):
        cid = lax.axis_index("c")
        sid = lax.axis_index("s")
        base = (cid * NS + sid) * SEG_W
        pltpu.sync_copy(psums_hbm.at[0, pl.ds(base, SEG_W)], a_v)
        pltpu.sync_copy(psums_hbm.at[1, pl.ds(base, SEG_W)], b_v)
        pltpu.sync_copy(pcnts_hbm.at[0, pl.ds(base, SEG_W)], ca_v)
        pltpu.sync_copy(pcnts_hbm.at[1, pl.ds(base, SEG_W)], cb_v)

        def body(s, carry):
            cnt = ca_v[s] + cb_v[s]
            inv = 1.0 / jnp.maximum(cnt, 1.0)
            for j in range(D // L):
                sl = pl.ds(j * L, L)
                a_v[s, sl] = (a_v[s, sl] + b_v[s, sl]) * inv
            return carry
        lax.fori_loop(0, SEG_W, body, 0)

        pltpu.sync_copy(a_v, out_hbm.at[pl.ds(base, SEG_W)])

    return k2(psums, pcnts)


def kernel(atom_fea, crystal_atom_idx):
    idx = crystal_atom_idx.astype(jnp.int32)
    idx3 = idx.reshape(-1, CHUNK // SUB, SUB)
    psums, pcnts = _partial_sums(atom_fea, idx3)
    return _combine(psums, pcnts)


# speedup vs baseline: 5.0926x; 5.0926x over previous
import functools

import jax
import jax.numpy as jnp
from jax import lax
from jax.experimental import pallas as pl
from jax.experimental.pallas import tpu as pltpu
from jax.experimental.pallas import tpu_sc as plsc

NUM_SEGMENTS = 4096
D = 128
L = 16
NC = 2
NS = 16
NW = NC * NS
CHUNK = 200
SUB = 100
SEG_T = NUM_SEGMENTS // NS
SEG_W = NUM_SEGMENTS // NW


def _partial_sums(atom_fea, idx3):
    n = atom_fea.shape[0]
    rows_per_w = n // NW
    n_chunks = rows_per_w // CHUNK
    mesh = plsc.VectorSubcoreMesh(core_axis_name="c", subcore_axis_name="s")

    @functools.partial(
        pl.kernel,
        mesh=mesh,
        out_type=[
            jax.ShapeDtypeStruct((NC, NUM_SEGMENTS, D), jnp.float32),
            jax.ShapeDtypeStruct((NC, NUM_SEGMENTS, D), jnp.float32),
        ],
        scratch_types=[
            pltpu.VMEM((CHUNK, D), jnp.float32),
            pltpu.VMEM((CHUNK // SUB, SUB), jnp.int32),
            pltpu.VMEM((SUB, D), jnp.float32),
            pltpu.VMEM_SHARED((NUM_SEGMENTS, D), jnp.float32),
            pltpu.VMEM_SHARED((NUM_SEGMENTS, D), jnp.float32),
        ],
    )
    def k1(fea_hbm, idx_hbm, psums_hbm, pcnts_hbm,
           rows_v, idx_v, ones_v, spm_sums, spm_cnts):
        cid = lax.axis_index("c")
        sid = lax.axis_index("s")
        wid = cid * NS + sid

        zeros16 = jnp.zeros((L,), jnp.float32)
        ones16 = jnp.ones((L,), jnp.float32)

        def zrow(r, carry):
            for j in range(D // L):
                rows_v[r, pl.ds(j * L, L)] = zeros16
            return carry
        lax.fori_loop(0, CHUNK, zrow, 0)

        def orow(r, carry):
            for j in range(D // L):
                ones_v[r, pl.ds(j * L, L)] = ones16
            return carry
        lax.fori_loop(0, SUB, orow, 0)

        half = SEG_T // 2
        for h in range(2):
            pltpu.sync_copy(
                rows_v.at[pl.ds(0, half)],
                spm_sums.at[pl.ds(sid * SEG_T + h * half, half)])
            pltpu.sync_copy(
                rows_v.at[pl.ds(0, half)],
                spm_cnts.at[pl.ds(sid * SEG_T + h * half, half)])
        plsc.subcore_barrier()

        base_row = wid * rows_per_w
        base_chunk = wid * n_chunks

        def body(i, carry):
            pltpu.sync_copy(fea_hbm.at[pl.ds(base_row + i * CHUNK, CHUNK)],
                            rows_v)
            pltpu.sync_copy(idx_hbm.at[base_chunk + i], idx_v)
            for j in range(CHUNK // SUB):
                pltpu.sync_copy(rows_v.at[pl.ds(j * SUB, SUB)],
                                spm_sums.at[idx_v.at[j]], add=True)
                pltpu.sync_copy(ones_v, spm_cnts.at[idx_v.at[j]], add=True)
            return carry
        lax.fori_loop(0, n_chunks, body, 0)

        plsc.subcore_barrier()

        pltpu.sync_copy(spm_sums.at[pl.ds(sid * SEG_T, SEG_T)],
                        psums_hbm.at[cid, pl.ds(sid * SEG_T, SEG_T)])
        pltpu.sync_copy(spm_cnts.at[pl.ds(sid * SEG_T, SEG_T)],
                        pcnts_hbm.at[cid, pl.ds(sid * SEG_T, SEG_T)])

    return k1(atom_fea, idx3)


def _combine(psums, pcnts):
    mesh = plsc.VectorSubcoreMesh(core_axis_name="c", subcore_axis_name="s")

    @functools.partial(
        pl.kernel,
        mesh=mesh,
        out_type=jax.ShapeDtypeStruct((NUM_SEGMENTS, D), jnp.float32),
        scratch_types=[
            pltpu.VMEM((SEG_W, D), jnp.float32),
            pltpu.VMEM((SEG_W, D), jnp.float32),
            pltpu.VMEM((SEG_W, D), jnp.float32),
            pltpu.VMEM((SEG_W, D), jnp.float32),
        ],
    )
    def k2(psums_hbm, pcnts_hbm, out_hbm, a_v, b_v, ca_v, cb_v):
        cid = lax.axis_index("c")
        sid = lax.axis_index("s")
        base = (cid * NS + sid) * SEG_W
        pltpu.sync_copy(psums_hbm.at[0, pl.ds(base, SEG_W)], a_v)
        pltpu.sync_copy(psums_hbm.at[1, pl.ds(base, SEG_W)], b_v)
        pltpu.sync_copy(pcnts_hbm.at[0, pl.ds(base, SEG_W)], ca_v)
        pltpu.sync_copy(pcnts_hbm.at[1, pl.ds(base, SEG_W)], cb_v)

        def body(s, carry):
            sl0 = pl.ds(0, L)
            cnt = ca_v[s, sl0] + cb_v[s, sl0]
            inv = 1.0 / jnp.maximum(cnt, 1.0)
            for j in range(D // L):
                sl = pl.ds(j * L, L)
                a_v[s, sl] = (a_v[s, sl] + b_v[s, sl]) * inv
            return carry
        lax.fori_loop(0, SEG_W, body, 0)

        pltpu.sync_copy(a_v, out_hbm.at[pl.ds(base, SEG_W)])

    return k2(psums, pcnts)


def kernel(atom_fea, crystal_atom_idx):
    idx = crystal_atom_idx.astype(jnp.int32)
    idx3 = idx.reshape(-1, CHUNK // SUB, SUB)
    psums, pcnts = _partial_sums(atom_fea, idx3)
    return _combine(psums, pcnts)


# speedup vs baseline: 6.5668x; 1.2895x over previous
import functools

import jax
import jax.numpy as jnp
from jax import lax
from jax.experimental import pallas as pl
from jax.experimental.pallas import tpu as pltpu
from jax.experimental.pallas import tpu_sc as plsc

NUM_SEGMENTS = 4096
D = 128
L = 16
NC = 2
NS = 16
NW = NC * NS
CHUNK = 200
SUB = 100
SEG_T = NUM_SEGMENTS // NS
SEG_W = NUM_SEGMENTS // NW


def _partial_sums(atom_fea, idx3):
    n = atom_fea.shape[0]
    rows_per_w = n // NW
    n_chunks = rows_per_w // CHUNK
    mesh = plsc.VectorSubcoreMesh(core_axis_name="c", subcore_axis_name="s")

    @functools.partial(
        pl.kernel,
        mesh=mesh,
        out_type=[
            jax.ShapeDtypeStruct((NC, NUM_SEGMENTS, D), jnp.float32),
            jax.ShapeDtypeStruct((NC, NUM_SEGMENTS, D), jnp.float32),
        ],
        scratch_types=[
            pltpu.VMEM((2, CHUNK, D), jnp.float32),
            pltpu.VMEM((2, CHUNK // SUB, SUB), jnp.int32),
            pltpu.VMEM((SUB, D), jnp.float32),
            pltpu.SemaphoreType.DMA,
            pltpu.SemaphoreType.DMA,
            pltpu.SemaphoreType.DMA,
            pltpu.SemaphoreType.DMA,
            pltpu.SemaphoreType.DMA,
            pltpu.SemaphoreType.DMA,
            pltpu.VMEM_SHARED((NUM_SEGMENTS, D), jnp.float32),
            pltpu.VMEM_SHARED((NUM_SEGMENTS, D), jnp.float32),
        ],
    )
    def k1(fea_hbm, idx_hbm, psums_hbm, pcnts_hbm,
           rows_v, idx_v, ones_v,
           sem_r0, sem_r1, sem_i0, sem_i1, sem_s0, sem_s1,
           spm_sums, spm_cnts):
        cid = lax.axis_index("c")
        sid = lax.axis_index("s")
        wid = cid * NS + sid

        zeros16 = jnp.zeros((L,), jnp.float32)
        ones16 = jnp.ones((L,), jnp.float32)

        half = SEG_T // 2

        def zrow(r, carry):
            for j in range(D // L):
                rows_v[0, r, pl.ds(j * L, L)] = zeros16
            return carry
        lax.fori_loop(0, half, zrow, 0)

        def orow(r, carry):
            for j in range(D // L):
                ones_v[r, pl.ds(j * L, L)] = ones16
            return carry
        lax.fori_loop(0, SUB, orow, 0)

        for h in range(2):
            pltpu.sync_copy(
                rows_v.at[0, pl.ds(0, half)],
                spm_sums.at[pl.ds(sid * SEG_T + h * half, half)])
            pltpu.sync_copy(
                rows_v.at[0, pl.ds(0, half)],
                spm_cnts.at[pl.ds(sid * SEG_T + h * half, half)])
        plsc.subcore_barrier()

        base_row = wid * rows_per_w
        base_chunk = wid * n_chunks
        sem_r = (sem_r0, sem_r1)
        sem_i = (sem_i0, sem_i1)
        sem_s = (sem_s0, sem_s1)

        def load(c, b):
            pltpu.async_copy(fea_hbm.at[pl.ds(base_row + c * CHUNK, CHUNK)],
                             rows_v.at[b], sem_r[b])
            pltpu.async_copy(idx_hbm.at[base_chunk + c], idx_v.at[b], sem_i[b])

        def wait_load(b):
            pltpu.make_async_copy(fea_hbm.at[pl.ds(0, CHUNK)],
                                  rows_v.at[b], sem_r[b]).wait()
            pltpu.make_async_copy(idx_hbm.at[0], idx_v.at[b], sem_i[b]).wait()

        for b in range(2):
            load(b, b)

        def body(i2, carry):
            for b in range(2):
                c = i2 * 2 + b
                wait_load(b)
                for j in range(CHUNK // SUB):
                    pltpu.sync_copy(rows_v.at[b, pl.ds(j * SUB, SUB)],
                                    spm_sums.at[idx_v.at[b, j]], add=True)
                    pltpu.sync_copy(ones_v, spm_cnts.at[idx_v.at[b, j]],
                                    add=True)

                @pl.when(c + 2 < n_chunks)
                def _():
                    load(c + 2, b)
            return carry
        lax.fori_loop(0, n_chunks // 2, body, 0)

        plsc.subcore_barrier()

        pltpu.sync_copy(spm_sums.at[pl.ds(sid * SEG_T, SEG_T)],
                        psums_hbm.at[cid, pl.ds(sid * SEG_T, SEG_T)])
        pltpu.sync_copy(spm_cnts.at[pl.ds(sid * SEG_T, SEG_T)],
                        pcnts_hbm.at[cid, pl.ds(sid * SEG_T, SEG_T)])

    return k1(atom_fea, idx3)


def _combine(psums, pcnts):
    mesh = plsc.VectorSubcoreMesh(core_axis_name="c", subcore_axis_name="s")

    @functools.partial(
        pl.kernel,
        mesh=mesh,
        out_type=jax.ShapeDtypeStruct((NUM_SEGMENTS, D), jnp.float32),
        scratch_types=[
            pltpu.VMEM((SEG_W, D), jnp.float32),
            pltpu.VMEM((SEG_W, D), jnp.float32),
            pltpu.VMEM((SEG_W, D), jnp.float32),
            pltpu.VMEM((SEG_W, D), jnp.float32),
        ],
    )
    def k2(psums_hbm, pcnts_hbm, out_hbm, a_v, b_v, ca_v, cb_v):
        cid = lax.axis_index("c")
        sid = lax.axis_index("s")
        base = (cid * NS + sid) * SEG_W
        pltpu.sync_copy(psums_hbm.at[0, pl.ds(base, SEG_W)], a_v)
        pltpu.sync_copy(psums_hbm.at[1, pl.ds(base, SEG_W)], b_v)
        pltpu.sync_copy(pcnts_hbm.at[0, pl.ds(base, SEG_W)], ca_v)
        pltpu.sync_copy(pcnts_hbm.at[1, pl.ds(base, SEG_W)], cb_v)

        def body(s, carry):
            sl0 = pl.ds(0, L)
            cnt = ca_v[s, sl0] + cb_v[s, sl0]
            inv = 1.0 / jnp.maximum(cnt, 1.0)
            for j in range(D // L):
                sl = pl.ds(j * L, L)
                a_v[s, sl] = (a_v[s, sl] + b_v[s, sl]) * inv
            return carry
        lax.fori_loop(0, SEG_W, body, 0)

        pltpu.sync_copy(a_v, out_hbm.at[pl.ds(base, SEG_W)])

    return k2(psums, pcnts)


def kernel(atom_fea, crystal_atom_idx):
    idx = crystal_atom_idx.astype(jnp.int32)
    idx3 = idx.reshape(-1, CHUNK // SUB, SUB)
    psums, pcnts = _partial_sums(atom_fea, idx3)
    return _combine(psums, pcnts)


# speedup vs baseline: 9.0696x; 1.3811x over previous
import functools

import jax
import jax.numpy as jnp
from jax import lax
from jax.experimental import pallas as pl
from jax.experimental.pallas import tpu as pltpu
from jax.experimental.pallas import tpu_sc as plsc

NUM_SEGMENTS = 4096
D = 128
L = 16
NC = 2
NS = 16
NW = NC * NS
CHUNK = 80
SUB = 80
NSUB = CHUNK // SUB
SEG_T = NUM_SEGMENTS // NS
SEG_W = NUM_SEGMENTS // NW
HALO = CHUNK + 16


def _partial_sums(atom_fea, idx3, ids_pad):
    n = atom_fea.shape[0]
    rows_per_w = n // NW
    n_chunks = rows_per_w // CHUNK
    mesh = plsc.VectorSubcoreMesh(core_axis_name="c", subcore_axis_name="s")

    @functools.partial(
        pl.kernel,
        mesh=mesh,
        compiler_params=pltpu.CompilerParams(needs_layout_passes=False),
        out_type=[
            jax.ShapeDtypeStruct((NC, NUM_SEGMENTS, D), jnp.float32),
            jax.ShapeDtypeStruct((NC, NS, SEG_T), jnp.float32),
        ],
        scratch_types=[
            pltpu.VMEM((2, CHUNK, D), jnp.float32),
            pltpu.VMEM((2, NSUB, SUB), jnp.int32),
            pltpu.VMEM((HALO,), jnp.int32),
            pltpu.VMEM((HALO,), jnp.int32),
            pltpu.VMEM((NUM_SEGMENTS,), jnp.float32),
            pltpu.VMEM((NUM_SEGMENTS,), jnp.float32),
            pltpu.VMEM((NS, SEG_T), jnp.float32),
            pltpu.SemaphoreType.DMA,
            pltpu.SemaphoreType.DMA,
            pltpu.SemaphoreType.DMA,
            pltpu.SemaphoreType.DMA,
            pltpu.SemaphoreType.DMA,
            pltpu.SemaphoreType.DMA,
            pltpu.VMEM_SHARED((NUM_SEGMENTS, D), jnp.float32),
            pltpu.VMEM_SHARED((NS, SEG_T * NS), jnp.float32),
        ],
    )
    def k1(fea_hbm, idx_hbm, ids_hbm, psums_hbm, pcnts_hbm,
           rows_v, idx_v, halo0_v, halo1_v, lo_v, hi_v, comb_v,
           sem_r0, sem_r1, sem_i0, sem_i1, sem_h0, sem_h1,
           spm_sums, spm_cstage):
        cid = lax.axis_index("c")
        sid = lax.axis_index("s")
        wid = cid * NS + sid

        zeros16 = jnp.zeros((L,), jnp.float32)
        half = SEG_T // 2

        def zrow(r, carry):
            for j in range(D // L):
                rows_v[0, r, pl.ds(j * L, L)] = zeros16
            return carry
        lax.fori_loop(0, half, zrow, 0)

        def zseg(r, carry):
            lo_v[pl.ds(r * L, L)] = zeros16
            hi_v[pl.ds(r * L, L)] = zeros16
            return carry
        lax.fori_loop(0, NUM_SEGMENTS // L, zseg, 0)

        for h in range(2):
            pltpu.sync_copy(
                rows_v.at[0, pl.ds(0, half)],
                spm_sums.at[pl.ds(sid * SEG_T + h * half, half)])
        plsc.subcore_barrier()

        halo_bufs = (halo0_v, halo1_v)
        base_row = wid * rows_per_w
        base_chunk = wid * n_chunks
        sem_r = (sem_r0, sem_r1)
        sem_i = (sem_i0, sem_i1)
        sem_h = (sem_h0, sem_h1)

        def load(c, b):
            pltpu.async_copy(fea_hbm.at[pl.ds(base_row + c * CHUNK, CHUNK)],
                             rows_v.at[b], sem_r[b])
            pltpu.async_copy(idx_hbm.at[base_chunk + c], idx_v.at[b], sem_i[b])
            pltpu.async_copy(ids_hbm.at[pl.ds(base_row + c * CHUNK, HALO)],
                             halo_bufs[b], sem_h[b])

        def wait_load(b):
            pltpu.make_async_copy(fea_hbm.at[pl.ds(0, CHUNK)],
                                  rows_v.at[b], sem_r[b]).wait()
            pltpu.make_async_copy(idx_hbm.at[0], idx_v.at[b], sem_i[b]).wait()
            pltpu.make_async_copy(ids_hbm.at[pl.ds(0, HALO)],
                                  halo_bufs[b], sem_h[b]).wait()

        for b in range(2):
            load(b, b)

        iota16 = lax.iota(jnp.int32, L)

        def process(c, b):
            wait_load(b)
            chunk_base = base_row + c * CHUNK
            for k in range(CHUNK // L):
                q = k * L
                cur = halo_bufs[b][pl.ds(8 + q, L)]
                prv = halo_bufs[b][pl.ds(7 + q, L)]
                nxt = halo_bufs[b][pl.ds(9 + q, L)]
                posf = (iota16 + (chunk_base + q)).astype(jnp.float32)
                plsc.store_scatter(lo_v, [cur], posf, mask=cur != prv)
                plsc.store_scatter(hi_v, [cur], posf + 1.0,
                                   mask=cur != nxt)
            for j in range(NSUB):
                pltpu.sync_copy(rows_v.at[b, pl.ds(j * SUB, SUB)],
                                spm_sums.at[idx_v.at[b, j]], add=True)

            @pl.when(c + 2 < n_chunks)
            def _():
                load(c + 2, b)

        def body(i2, carry):
            for b in range(2):
                process(i2 * 2 + b, b)
            return carry
        lax.fori_loop(0, n_chunks // 2, body, 0)
        if n_chunks % 2:
            process(n_chunks - 1, (n_chunks - 1) % 2)

        def diff(r, carry):
            sl = pl.ds(r * L, L)
            hi_v[sl] = hi_v[sl] - lo_v[sl]
            return carry
        lax.fori_loop(0, NUM_SEGMENTS // L, diff, 0)
        pltpu.sync_copy(hi_v, spm_cstage.at[sid])

        plsc.subcore_barrier()

        pltpu.sync_copy(spm_sums.at[pl.ds(sid * SEG_T, SEG_T)],
                        psums_hbm.at[cid, pl.ds(sid * SEG_T, SEG_T)])

        pltpu.sync_copy(spm_cstage.at[:, pl.ds(sid * SEG_T, SEG_T)], comb_v)

        def csum(r, carry):
            sl = pl.ds(r * L, L)
            acc = comb_v[0, sl]
            for t in range(1, NS):
                acc = acc + comb_v[t, sl]
            comb_v[0, sl] = acc
            return carry
        lax.fori_loop(0, SEG_T // L, csum, 0)
        pltpu.sync_copy(comb_v.at[0], pcnts_hbm.at[cid, sid])

    return k1(atom_fea, idx3, ids_pad)


def _combine(psums, pcnts):
    mesh = plsc.VectorSubcoreMesh(core_axis_name="c", subcore_axis_name="s")

    @functools.partial(
        pl.kernel,
        mesh=mesh,
        compiler_params=pltpu.CompilerParams(needs_layout_passes=False),
        out_type=jax.ShapeDtypeStruct((NUM_SEGMENTS, D), jnp.float32),
        scratch_types=[
            pltpu.VMEM((SEG_W, D), jnp.float32),
            pltpu.VMEM((SEG_W, D), jnp.float32),
            pltpu.VMEM((SEG_W,), jnp.float32),
            pltpu.VMEM((SEG_W,), jnp.float32),
            pltpu.VMEM((SEG_W * L,), jnp.float32),
        ],
    )
    def k2(psums_hbm, pcnts_hbm, out_hbm, a_v, b_v, ca_v, cb_v, rep_v):
        cid = lax.axis_index("c")
        sid = lax.axis_index("s")
        base = (cid * NS + sid) * SEG_W
        pltpu.sync_copy(psums_hbm.at[0, pl.ds(base, SEG_W)], a_v)
        pltpu.sync_copy(psums_hbm.at[1, pl.ds(base, SEG_W)], b_v)
        pltpu.sync_copy(pcnts_hbm.at[0, pl.ds(base, SEG_W)], ca_v)
        pltpu.sync_copy(pcnts_hbm.at[1, pl.ds(base, SEG_W)], cb_v)

        iota16 = lax.iota(jnp.int32, L)
        for g in range(SEG_W // L):
            sl = pl.ds(g * L, L)
            inv = 1.0 / jnp.maximum(ca_v[sl] + cb_v[sl], 1.0)
            for r in range(L):
                plsc.store_scatter(
                    rep_v, [iota16 * L + (g * L * L + r)], inv)

        def body(s, carry):
            inv = rep_v[pl.ds(s * L, L)]
            for j in range(D // L):
                sl = pl.ds(j * L, L)
                a_v[s, sl] = (a_v[s, sl] + b_v[s, sl]) * inv
            return carry
        lax.fori_loop(0, SEG_W, body, 0)

        pltpu.sync_copy(a_v, out_hbm.at[pl.ds(base, SEG_W)])

    return k2(psums, pcnts)


def kernel(atom_fea, crystal_atom_idx):
    idx = crystal_atom_idx.astype(jnp.int32)
    idx3 = idx.reshape(-1, NSUB, SUB)
    pad = jnp.full((8,), -1, jnp.int32)
    ids_pad = jnp.concatenate([pad, idx, pad])
    psums, pcnts = _partial_sums(atom_fea, idx3, ids_pad)
    return _combine(psums, pcnts.reshape(NC, NUM_SEGMENTS))


# speedup vs baseline: 10.2341x; 1.1284x over previous
import functools

import jax
import jax.numpy as jnp
from jax import lax
from jax.experimental import pallas as pl
from jax.experimental.pallas import tpu as pltpu
from jax.experimental.pallas import tpu_sc as plsc

NUM_SEGMENTS = 4096
D = 128
L = 16
NC = 2
NS = 16
NW = NC * NS
CHUNK = 128
RW_BASE = 9984
SEG_T = NUM_SEGMENTS // NS
SEG_W = NUM_SEGMENTS // NW
HALO = CHUNK + 16


def _partial_sums(atom_fea, idx2, ids_pad):
    n = atom_fea.shape[0]
    extra = (n - NW * RW_BASE) // CHUNK
    mesh = plsc.VectorSubcoreMesh(core_axis_name="c", subcore_axis_name="s")

    @functools.partial(
        pl.kernel,
        mesh=mesh,
        compiler_params=pltpu.CompilerParams(needs_layout_passes=False),
        out_type=[
            jax.ShapeDtypeStruct((NC, NUM_SEGMENTS, D), jnp.float32),
            jax.ShapeDtypeStruct((NC, NS, SEG_T), jnp.float32),
        ],
        scratch_types=[
            pltpu.VMEM((4, CHUNK, D), jnp.float32),
            pltpu.VMEM((4, CHUNK), jnp.int32),
            pltpu.VMEM((HALO,), jnp.int32),
            pltpu.VMEM((HALO,), jnp.int32),
            pltpu.VMEM((HALO,), jnp.int32),
            pltpu.VMEM((HALO,), jnp.int32),
            pltpu.VMEM((NUM_SEGMENTS,), jnp.float32),
            pltpu.VMEM((NUM_SEGMENTS,), jnp.float32),
            pltpu.VMEM((NS, SEG_T), jnp.float32),
            [pltpu.SemaphoreType.DMA] * 4,
            [pltpu.SemaphoreType.DMA] * 4,
            [pltpu.SemaphoreType.DMA] * 4,
            [pltpu.SemaphoreType.DMA] * 4,
            pltpu.VMEM_SHARED((NUM_SEGMENTS, D), jnp.float32),
            pltpu.VMEM_SHARED((NS, SEG_T * NS), jnp.float32),
        ],
    )
    def k1(fea_hbm, idx_hbm, ids_hbm, psums_hbm, pcnts_hbm,
           rows_v, idx_v, halo0_v, halo1_v, halo2_v, halo3_v,
           lo_v, hi_v, comb_v,
           sem_r, sem_i, sem_h, sem_s,
           spm_sums, spm_cstage):
        cid = lax.axis_index("c")
        sid = lax.axis_index("s")
        wid = cid * NS + sid

        zeros16 = jnp.zeros((L,), jnp.float32)
        half = SEG_T // 2

        def zrow(r, carry):
            for j in range(D // L):
                rows_v[0, r, pl.ds(j * L, L)] = zeros16
            return carry
        lax.fori_loop(0, half, zrow, 0)

        def zseg(r, carry):
            lo_v[pl.ds(r * L, L)] = zeros16
            hi_v[pl.ds(r * L, L)] = zeros16
            return carry
        lax.fori_loop(0, NUM_SEGMENTS // L, zseg, 0)

        for h in range(2):
            pltpu.sync_copy(
                rows_v.at[0, pl.ds(0, half)],
                spm_sums.at[pl.ds(sid * SEG_T + h * half, half)])
        plsc.subcore_barrier()

        halo_bufs = (halo0_v, halo1_v, halo2_v, halo3_v)
        base_row = wid * RW_BASE + jnp.minimum(wid, extra) * CHUNK
        base_chunk = base_row // CHUNK
        n_chunks = RW_BASE // CHUNK + jnp.where(wid < extra, 1, 0)

        def load(c, b):
            pltpu.async_copy(fea_hbm.at[pl.ds(base_row + c * CHUNK, CHUNK)],
                             rows_v.at[b], sem_r[b])
            pltpu.async_copy(idx_hbm.at[base_chunk + c], idx_v.at[b], sem_i[b])
            pltpu.async_copy(ids_hbm.at[pl.ds(base_row + c * CHUNK, HALO)],
                             halo_bufs[b], sem_h[b])

        def wait_load(b):
            pltpu.make_async_copy(fea_hbm.at[pl.ds(0, CHUNK)],
                                  rows_v.at[b], sem_r[b]).wait()
            pltpu.make_async_copy(idx_hbm.at[0], idx_v.at[b], sem_i[b]).wait()
            pltpu.make_async_copy(ids_hbm.at[pl.ds(0, HALO)],
                                  halo_bufs[b], sem_h[b]).wait()

        for b in range(2):
            load(b, b)

        iota16 = lax.iota(jnp.int32, L)

        def scat_wait(b):
            pltpu.make_async_copy(rows_v.at[b],
                                  spm_sums.at[idx_v.at[b]],
                                  sem_s[b]).wait()

        def process(c, b):
            b2 = (b + 2) % 4

            @pl.when(c >= 2)
            def _():
                scat_wait(b2)

            @pl.when(c + 2 < n_chunks)
            def _():
                load(c + 2, b2)

            wait_load(b)
            scat = pltpu.make_async_copy(rows_v.at[b],
                                         spm_sums.at[idx_v.at[b]],
                                         sem_s[b])
            scat.start(add=True)
            chunk_base = base_row + c * CHUNK
            for k in range(CHUNK // L):
                q = k * L
                cur = halo_bufs[b][pl.ds(8 + q, L)]
                prv = halo_bufs[b][pl.ds(7 + q, L)]
                nxt = halo_bufs[b][pl.ds(9 + q, L)]
                posf = (iota16 + (chunk_base + q)).astype(jnp.float32)
                plsc.store_scatter(lo_v, [cur], posf, mask=cur != prv)
                plsc.store_scatter(hi_v, [cur], posf + 1.0,
                                   mask=cur != nxt)

        def body(c, carry):
            for b in range(4):
                @pl.when(c % 4 == b)
                def _(b=b):
                    process(c, b)
            return carry
        lax.fori_loop(0, n_chunks, body, 0)

        last1 = (n_chunks - 1) % 4
        last2 = (n_chunks - 2) % 4
        for b in range(4):
            @pl.when((last1 == b) | (last2 == b))
            def _(b=b):
                scat_wait(b)

        def diff(r, carry):
            sl = pl.ds(r * L, L)
            hi_v[sl] = hi_v[sl] - lo_v[sl]
            return carry
        lax.fori_loop(0, NUM_SEGMENTS // L, diff, 0)
        pltpu.sync_copy(hi_v, spm_cstage.at[sid])

        plsc.subcore_barrier()

        pltpu.sync_copy(spm_sums.at[pl.ds(sid * SEG_T, SEG_T)],
                        psums_hbm.at[cid, pl.ds(sid * SEG_T, SEG_T)])

        pltpu.sync_copy(spm_cstage.at[:, pl.ds(sid * SEG_T, SEG_T)], comb_v)

        def csum(r, carry):
            sl = pl.ds(r * L, L)
            acc = comb_v[0, sl]
            for t in range(1, NS):
                acc = acc + comb_v[t, sl]
            comb_v[0, sl] = acc
            return carry
        lax.fori_loop(0, SEG_T // L, csum, 0)
        pltpu.sync_copy(comb_v.at[0], pcnts_hbm.at[cid, sid])

    return k1(atom_fea, idx2, ids_pad)


def _combine(psums, pcnts):
    mesh = plsc.VectorSubcoreMesh(core_axis_name="c", subcore_axis_name="s")

    @functools.partial(
        pl.kernel,
        mesh=mesh,
        compiler_params=pltpu.CompilerParams(needs_layout_passes=False),
        out_type=jax.ShapeDtypeStruct((NUM_SEGMENTS, D), jnp.float32),
        scratch_types=[
            pltpu.VMEM((SEG_W, D), jnp.float32),
            pltpu.VMEM((SEG_W, D), jnp.float32),
            pltpu.VMEM((SEG_W,), jnp.float32),
            pltpu.VMEM((SEG_W,), jnp.float32),
            pltpu.VMEM((SEG_W * L,), jnp.float32),
        ],
    )
    def k2(psums_hbm, pcnts_hbm, out_hbm, a_v, b_v, ca_v, cb_v, rep_v):
        cid = lax.axis_index("c")
        sid = lax.axis_index("s")
        base = (cid * NS + sid) * SEG_W
        pltpu.sync_copy(psums_hbm.at[0, pl.ds(base, SEG_W)], a_v)
        pltpu.sync_copy(psums_hbm.at[1, pl.ds(base, SEG_W)], b_v)
        pltpu.sync_copy(pcnts_hbm.at[0, pl.ds(base, SEG_W)], ca_v)
        pltpu.sync_copy(pcnts_hbm.at[1, pl.ds(base, SEG_W)], cb_v)

        iota16 = lax.iota(jnp.int32, L)
        for g in range(SEG_W // L):
            sl = pl.ds(g * L, L)
            inv = 1.0 / jnp.maximum(ca_v[sl] + cb_v[sl], 1.0)
            for r in range(L):
                plsc.store_scatter(
                    rep_v, [iota16 * L + (g * L * L + r)], inv)

        def body(s, carry):
            inv = rep_v[pl.ds(s * L, L)]
            for j in range(D // L):
                sl = pl.ds(j * L, L)
                a_v[s, sl] = (a_v[s, sl] + b_v[s, sl]) * inv
            return carry
        lax.fori_loop(0, SEG_W, body, 0)

        pltpu.sync_copy(a_v, out_hbm.at[pl.ds(base, SEG_W)])

    return k2(psums, pcnts)


def kernel(atom_fea, crystal_atom_idx):
    idx = crystal_atom_idx.astype(jnp.int32)
    idx2 = idx.reshape(-1, CHUNK)
    pad_front = jnp.full((8,), -1, jnp.int32)
    pad_back = jnp.full((HALO - CHUNK - 8,) if HALO - CHUNK - 8 > 8
                        else (8,), -1, jnp.int32)
    ids_pad = jnp.concatenate([pad_front, idx, pad_back])
    psums, pcnts = _partial_sums(atom_fea, idx2, ids_pad)
    return _combine(psums, pcnts.reshape(NC, NUM_SEGMENTS))
